# baseline (speedup 1.0000x reference)
import jax
import jax.numpy as jnp
from jax import lax
from jax.experimental import pallas as pl
from jax.experimental.pallas import tpu as pltpu
from jax.experimental.pallas import tpu_sc as plsc

VOCAB = 100000
EMB = 64
MAX_NORM = 1.0
B = 1024
CTX = 50

NUM_CORES = 2
NUM_SUBCORES = 16
NUM_WORKERS = NUM_CORES * NUM_SUBCORES
BPW = B // NUM_WORKERS
LANES = 16
EV = EMB // LANES
PAD = 2 * EMB

VP = 4096


def _sp_body(wt_ref, o_ref):
    w = wt_ref[...]
    ss = lax.dot_general(jnp.ones((1, EMB), jnp.float32), w * w,
                         dimension_numbers=(((1,), (0,)), ((), ())),
                         preferred_element_type=jnp.float32)
    norm = jnp.sqrt(ss)
    scale = jnp.minimum(1.0, MAX_NORM / jnp.maximum(norm, 1e-7)) * (1.0 / CTX)
    scaled = w * scale
    scaled_t = lax.dot_general(scaled, jnp.eye(EMB, dtype=jnp.float32),
                               dimension_numbers=(((0,), (0,)), ((), ())),
                               preferred_element_type=jnp.float32)
    o_ref[:, pl.ds(0, EMB)] = scaled_t


def _scalepack(wt):
    return pl.pallas_call(
        _sp_body,
        grid=(pl.cdiv(VOCAB, VP),),
        in_specs=[pl.BlockSpec((EMB, VP), lambda i: (0, i))],
        out_specs=pl.BlockSpec((VP, PAD), lambda i: (i, 0)),
        out_shape=jax.ShapeDtypeStruct((VOCAB, PAD), jnp.float32),
    )(wt)


IPG = 2
NG = BPW // IPG


NBUF = 6


def _pool_body(idx_hbm, ptab_hbm, x_hbm, idx_v, rows_a, rows_b, rows_c,
               rows_d, rows_e, rows_f, x_v, sem_a, sem_b, sem_c, sem_d,
               sem_e, sem_f):
    wid = lax.axis_index("s") * NUM_CORES + lax.axis_index("c")
    b0 = wid * BPW
    g0 = wid * NG
    pltpu.sync_copy(idx_hbm.at[pl.ds(g0, NG), :], idx_v)

    bufs = (rows_a, rows_b, rows_c, rows_d, rows_e, rows_f)
    sems = (sem_a, sem_b, sem_c, sem_d, sem_e, sem_f)
    copies = [None] * NBUF

    def start(g):
        j = g % NBUF
        copies[j] = pltpu.async_copy(ptab_hbm.at[idx_v.at[g]], bufs[j],
                                     sems[j])

    for g in range(NBUF - 1):
        start(g)

    for g in range(NG):
        if g + NBUF - 1 < NG:
            start(g + NBUF - 1)
        copies[g % NBUF].wait()
        rows = bufs[g % NBUF]

        for s in range(IPG):
            def row_step(r, accs, rows=rows, s=s):
                return tuple(a + rows[s * CTX + r, pl.ds(k * LANES, LANES)]
                             for k, a in enumerate(accs))

            zero = jnp.zeros((LANES,), jnp.float32)
            accs = lax.fori_loop(0, CTX, row_step, (zero,) * EV)
            for k in range(EV):
                x_v[g * IPG + s, pl.ds(k * LANES, LANES)] = accs[k]

    pltpu.sync_copy(x_v, x_hbm.at[pl.ds(b0, BPW), :])


def _pool(idx2d, ptab):
    mesh = plsc.VectorSubcoreMesh(core_axis_name="c", subcore_axis_name="s")
    return pl.kernel(
        _pool_body,
        out_type=jax.ShapeDtypeStruct((B, EMB), jnp.float32),
        mesh=mesh,
        scratch_types=[
            pltpu.VMEM((NG, IPG * CTX), jnp.int32),
            pltpu.VMEM((IPG * CTX, PAD), jnp.float32),
            pltpu.VMEM((IPG * CTX, PAD), jnp.float32),
            pltpu.VMEM((IPG * CTX, PAD), jnp.float32),
            pltpu.VMEM((IPG * CTX, PAD), jnp.float32),
            pltpu.VMEM((IPG * CTX, PAD), jnp.float32),
            pltpu.VMEM((IPG * CTX, PAD), jnp.float32),
            pltpu.VMEM((BPW, EMB), jnp.float32),
            pltpu.SemaphoreType.DMA,
            pltpu.SemaphoreType.DMA,
            pltpu.SemaphoreType.DMA,
            pltpu.SemaphoreType.DMA,
            pltpu.SemaphoreType.DMA,
            pltpu.SemaphoreType.DMA,
        ],
    )(idx2d, ptab)


V_TILE = 4096


def _mm_body(x_ref, w_ref, b_ref, o_ref):
    acc = lax.dot_general(
        w_ref[...], x_ref[...],
        dimension_numbers=(((0,), (1,)), ((), ())),
        preferred_element_type=jnp.float32)
    o_ref[...] = acc + jnp.transpose(b_ref[...])


def _project(x, wt, b2):
    grid = (pl.cdiv(VOCAB, V_TILE),)
    return pl.pallas_call(
        _mm_body,
        grid=grid,
        in_specs=[
            pl.BlockSpec((B, EMB), lambda i: (0, 0)),
            pl.BlockSpec((EMB, V_TILE), lambda i: (0, i)),
            pl.BlockSpec((1, V_TILE), lambda i: (0, i)),
        ],
        out_specs=pl.BlockSpec((V_TILE, B), lambda i: (i, 0)),
        out_shape=jax.ShapeDtypeStruct((VOCAB, B), jnp.float32),
    )(x, wt, b2)


def kernel(inputs_, emb_table, W, b):
    ptab = _scalepack(emb_table.T)
    x = _pool(inputs_.reshape(B // IPG, IPG * CTX), ptab)
    return _project(x, W.T, b.reshape(1, VOCAB)).T


# speedup vs baseline: 1.0403x; 1.0403x over previous
import jax
import jax.numpy as jnp
from jax import lax
from jax.experimental import pallas as pl
from jax.experimental.pallas import tpu as pltpu
from jax.experimental.pallas import tpu_sc as plsc

VOCAB = 100000
EMB = 64
MAX_NORM = 1.0
B = 1024
CTX = 50

NUM_CORES = 2
NUM_SUBCORES = 16
NUM_WORKERS = NUM_CORES * NUM_SUBCORES
BPW = B // NUM_WORKERS
LANES = 16
EV = EMB // LANES
PAD = 2 * EMB

VP = 8192


def _sp_body(wt_ref, o_ref):
    w = wt_ref[...]
    ss = lax.dot_general(jnp.ones((1, EMB), jnp.float32), w * w,
                         dimension_numbers=(((1,), (0,)), ((), ())),
                         preferred_element_type=jnp.float32)
    norm = jnp.sqrt(ss)
    scale = jnp.minimum(1.0, MAX_NORM / jnp.maximum(norm, 1e-7)) * (1.0 / CTX)
    scaled = w * scale
    scaled_t = lax.dot_general(scaled, jnp.eye(EMB, dtype=jnp.float32),
                               dimension_numbers=(((0,), (0,)), ((), ())),
                               preferred_element_type=jnp.float32)
    o_ref[:, pl.ds(0, EMB)] = scaled_t


def _scalepack(wt):
    return pl.pallas_call(
        _sp_body,
        grid=(pl.cdiv(VOCAB, VP),),
        in_specs=[pl.BlockSpec((EMB, VP), lambda i: (0, i))],
        out_specs=pl.BlockSpec((VP, PAD), lambda i: (i, 0)),
        out_shape=jax.ShapeDtypeStruct((VOCAB, PAD), jnp.float32),
    )(wt)


IPG = 2
NG = BPW // IPG


NBUF = 6


def _pool_body(idx_hbm, ptab_hbm, x_hbm, idx_v, rows_a, rows_b, rows_c,
               rows_d, rows_e, rows_f, x_v, sem_a, sem_b, sem_c, sem_d,
               sem_e, sem_f):
    wid = lax.axis_index("s") * NUM_CORES + lax.axis_index("c")
    b0 = wid * BPW
    g0 = wid * NG
    pltpu.sync_copy(idx_hbm.at[pl.ds(g0, NG), :], idx_v)

    bufs = (rows_a, rows_b, rows_c, rows_d, rows_e, rows_f)
    sems = (sem_a, sem_b, sem_c, sem_d, sem_e, sem_f)
    copies = [None] * NBUF

    def start(g):
        j = g % NBUF
        copies[j] = pltpu.async_copy(ptab_hbm.at[idx_v.at[g]], bufs[j],
                                     sems[j])

    for g in range(NBUF - 1):
        start(g)

    for g in range(NG):
        if g + NBUF - 1 < NG:
            start(g + NBUF - 1)
        copies[g % NBUF].wait()
        rows = bufs[g % NBUF]

        for s in range(IPG):
            def row_step(r, accs, rows=rows, s=s):
                return tuple(a + rows[s * CTX + r, pl.ds(k * LANES, LANES)]
                             for k, a in enumerate(accs))

            zero = jnp.zeros((LANES,), jnp.float32)
            accs = lax.fori_loop(0, CTX, row_step, (zero,) * EV)
            for k in range(EV):
                x_v[g * IPG + s, pl.ds(k * LANES, LANES)] = accs[k]

    pltpu.sync_copy(x_v, x_hbm.at[pl.ds(b0, BPW), :])


def _pool(idx2d, ptab):
    mesh = plsc.VectorSubcoreMesh(core_axis_name="c", subcore_axis_name="s")
    return pl.kernel(
        _pool_body,
        out_type=jax.ShapeDtypeStruct((B, EMB), jnp.float32),
        mesh=mesh,
        scratch_types=[
            pltpu.VMEM((NG, IPG * CTX), jnp.int32),
            pltpu.VMEM((IPG * CTX, PAD), jnp.float32),
            pltpu.VMEM((IPG * CTX, PAD), jnp.float32),
            pltpu.VMEM((IPG * CTX, PAD), jnp.float32),
            pltpu.VMEM((IPG * CTX, PAD), jnp.float32),
            pltpu.VMEM((IPG * CTX, PAD), jnp.float32),
            pltpu.VMEM((IPG * CTX, PAD), jnp.float32),
            pltpu.VMEM((BPW, EMB), jnp.float32),
            pltpu.SemaphoreType.DMA,
            pltpu.SemaphoreType.DMA,
            pltpu.SemaphoreType.DMA,
            pltpu.SemaphoreType.DMA,
            pltpu.SemaphoreType.DMA,
            pltpu.SemaphoreType.DMA,
        ],
    )(idx2d, ptab)


V_TILE = 4096


def _mm_body(x_ref, w_ref, b_ref, o_ref):
    acc = lax.dot_general(
        w_ref[...], x_ref[...],
        dimension_numbers=(((0,), (1,)), ((), ())),
        preferred_element_type=jnp.float32)
    o_ref[...] = acc + jnp.transpose(b_ref[...])


def _project(x, wt, b2):
    grid = (pl.cdiv(VOCAB, V_TILE),)
    return pl.pallas_call(
        _mm_body,
        grid=grid,
        in_specs=[
            pl.BlockSpec((B, EMB), lambda i: (0, 0)),
            pl.BlockSpec((EMB, V_TILE), lambda i: (0, i)),
            pl.BlockSpec((1, V_TILE), lambda i: (0, i)),
        ],
        out_specs=pl.BlockSpec((V_TILE, B), lambda i: (i, 0)),
        out_shape=jax.ShapeDtypeStruct((VOCAB, B), jnp.float32),
    )(x, wt, b2)


def kernel(inputs_, emb_table, W, b):
    ptab = _scalepack(emb_table.T)
    x = _pool(inputs_.reshape(B // IPG, IPG * CTX), ptab)
    return _project(x, W.T, b.reshape(1, VOCAB)).T
